# baseline (speedup 1.0000x reference)
import functools

import jax
import jax.numpy as jnp
from jax import lax
from jax.experimental import pallas as pl
from jax.experimental.pallas import tpu as pltpu
from jax.experimental.pallas import tpu_sc as plsc

_N_NODES = 10000
_N_EDGES = 320000
_N_IN = 128
_N_HID = 128
_E_IN = 16
_EF = 4
_E_HID = 128

_NPC = 2
_EP = _N_EDGES // _NPC
_NW = 32
_C = 40
_E_PER_W = _EP // _NW
_N_CHUNKS = _E_PER_W // _C

_HI = jax.lax.Precision.HIGHEST


def _prep_body(x_ref, w1t_ref, ei_ref, h_ref, src_ref, dst_ref):
    h_ref[...] = jnp.dot(x_ref[...], w1t_ref[...],
                         preferred_element_type=jnp.float32, precision=_HI)
    start = jnp.min(ei_ref[0, :])
    src_ref[...] = jnp.clip(ei_ref[0, :] - start, 0, _N_NODES - 1)
    dst_ref[...] = jnp.clip(ei_ref[1, :] - start, 0, _N_NODES - 1)


def _prep(x, w1t, ei):
    return pl.pallas_call(
        _prep_body,
        out_shape=(
            jax.ShapeDtypeStruct((_N_NODES, _N_HID), jnp.float32),
            jax.ShapeDtypeStruct((_N_EDGES,), jnp.int32),
            jax.ShapeDtypeStruct((_N_EDGES,), jnp.int32),
        ),
    )(x, w1t, ei)


_NBUF = 3
_AHEAD = 1


def _make_gather(piece):
    @functools.partial(
        pl.kernel,
        mesh=plsc.VectorSubcoreMesh(core_axis_name="c", subcore_axis_name="s"),
        out_type=jax.ShapeDtypeStruct((_EP, _N_HID), jnp.float32),
        scratch_types=[
            pltpu.VMEM((_E_PER_W,), jnp.int32),
            pltpu.VMEM((_E_PER_W,), jnp.int32),
            pltpu.VMEM((_NBUF, _C, _N_HID), jnp.float32),
            pltpu.VMEM((_NBUF, _C, _N_HID), jnp.float32),
            pltpu.VMEM_SHARED((_N_NODES, _N_HID), jnp.float32),
            pltpu.SemaphoreType.DMA((_NBUF,)),
            pltpu.SemaphoreType.DMA((_NBUF,)),
        ],
    )
    def _gather_diff(h_hbm, src_hbm, dst_hbm, out_hbm, isl, idl, rs, rd, hsh, gsem, osem):
        sid = lax.axis_index("s")
        wid = sid * 2 + lax.axis_index("c")
        obase = wid * _E_PER_W
        ebase = piece * _EP + obase

        @pl.when(sid < 15)
        def _():
            pltpu.sync_copy(h_hbm.at[pl.ds(sid * 640, 640)], hsh.at[pl.ds(sid * 640, 640)])

        @pl.when(sid == 15)
        def _():
            pltpu.sync_copy(h_hbm.at[pl.ds(9600, 400)], hsh.at[pl.ds(9600, 400)])

        pltpu.sync_copy(src_hbm.at[pl.ds(ebase, _E_PER_W)], isl)
        pltpu.sync_copy(dst_hbm.at[pl.ds(ebase, _E_PER_W)], idl)
        plsc.subcore_barrier()

        def start_gather(i, slot):
            pltpu.async_copy(hsh.at[isl.at[pl.ds(i * _C, _C)]], rs.at[slot], gsem.at[slot])
            pltpu.async_copy(hsh.at[idl.at[pl.ds(i * _C, _C)]], rd.at[slot], gsem.at[slot])

        def wait_gather(slot):
            pltpu.make_async_copy(h_hbm.at[pl.ds(0, _C)], rs.at[slot], gsem.at[slot]).wait()
            pltpu.make_async_copy(h_hbm.at[pl.ds(0, _C)], rd.at[slot], gsem.at[slot]).wait()

        def drain_scatter(slot):
            pltpu.make_async_copy(rs.at[slot], out_hbm.at[pl.ds(0, _C)], osem.at[slot]).wait()

        for j in range(_AHEAD):
            start_gather(j, j)

        def body(i, carry):
            p = lax.rem(i, _NBUF)

            @pl.when(i + _AHEAD < _N_CHUNKS)
            def _():
                q = lax.rem(i + _AHEAD, _NBUF)

                @pl.when(i >= _NBUF - _AHEAD)
                def _():
                    drain_scatter(q)

                start_gather(i + _AHEAD, q)

            wait_gather(p)

            def row_body(r, carry2):
                for k in range(_N_HID // 16):
                    sl = pl.ds(k * 16, 16)
                    rs[p, r, sl] = jnp.maximum(rs[p, r, sl] - rd[p, r, sl], 0.0)
                return carry2

            lax.fori_loop(0, _C, row_body, 0)

            pltpu.async_copy(rs.at[p], out_hbm.at[pl.ds(obase + i * _C, _C)], osem.at[p])
            return carry

        lax.fori_loop(0, _N_CHUNKS, body, 0)
        for slot in range(_NBUF):
            drain_scatter(slot)

    return _gather_diff


_gather_piece = [_make_gather(p) for p in range(_NPC)]


_BLK = 3200
_BPP = _EP // _BLK


def _out_body(xem_ref, ea_ref, ef_ref, wem_ref, wea_ref, wef_ref, b2_ref, out_ref):
    acc = jnp.dot(xem_ref[...].astype(jnp.bfloat16), wem_ref[...],
                  preferred_element_type=jnp.float32)
    acc = acc + jnp.dot(ea_ref[...].astype(jnp.bfloat16), wea_ref[...],
                        preferred_element_type=jnp.float32)
    acc = acc + jnp.dot(ef_ref[...].astype(jnp.bfloat16), wef_ref[...],
                        preferred_element_type=jnp.float32)
    out_ref[...] = acc + b2_ref[...]


def _acc_body(prev_ref, xem_ref, ea_ref, ef_ref, wem_ref, wea_ref, wef_ref,
              b2_ref, out_ref):
    del prev_ref
    _out_body(xem_ref, ea_ref, ef_ref, wem_ref, wea_ref, wef_ref, b2_ref, out_ref)


def _final_piece(piece, xem, ea, ef, wem, wea, wef, b2row, prev=None):
    off = piece * _BPP
    specs = [
        pl.BlockSpec((_BLK, _N_HID), lambda i: (i, 0)),
        pl.BlockSpec((_BLK, _E_IN), lambda i, o=off: (i + o, 0)),
        pl.BlockSpec((_BLK, _EF), lambda i, o=off: (i + o, 0)),
        pl.BlockSpec((_N_HID, _E_HID), lambda i: (0, 0)),
        pl.BlockSpec((_E_IN, _E_HID), lambda i: (0, 0)),
        pl.BlockSpec((_EF, _E_HID), lambda i: (0, 0)),
        pl.BlockSpec((1, _E_HID), lambda i: (0, 0)),
    ]
    out_spec = pl.BlockSpec((_BLK, _E_HID), lambda i, o=off: (i + o, 0))
    out_shape = jax.ShapeDtypeStruct((_N_EDGES, _E_HID), jnp.float32)
    if prev is None:
        return pl.pallas_call(
            _out_body, grid=(_BPP,), in_specs=specs, out_specs=out_spec,
            out_shape=out_shape,
        )(xem, ea, ef, wem, wea, wef, b2row)
    prev_spec = pl.BlockSpec((8, _E_HID), lambda i: (0, 0))
    return pl.pallas_call(
        _acc_body, grid=(_BPP,), in_specs=[prev_spec] + specs,
        out_specs=out_spec, out_shape=out_shape,
        input_output_aliases={0: 0},
    )(prev, xem, ea, ef, wem, wea, wef, b2row)


def kernel(x, edge_index, edge_f, edge_attr, W1, b1, W2, b2):
    del b1
    h, src, dst = _prep(x, W1.T, edge_index)
    wem = W2[:, :_N_HID].T.astype(jnp.bfloat16)
    wea = W2[:, _N_HID:_N_HID + _E_IN].T.astype(jnp.bfloat16)
    wef = W2[:, _N_HID + _E_IN:].T.astype(jnp.bfloat16)
    b2row = b2.reshape(1, _E_HID)
    xem0 = _gather_piece[0](h, src, dst)
    xem1 = _gather_piece[1](h, src, dst)
    out = _final_piece(0, xem0, edge_attr, edge_f, wem, wea, wef, b2row)
    out = _final_piece(1, xem1, edge_attr, edge_f, wem, wea, wef, b2row, prev=out)
    return out


# speedup vs baseline: 1.0391x; 1.0391x over previous
import functools

import jax
import jax.numpy as jnp
from jax import lax
from jax.experimental import pallas as pl
from jax.experimental.pallas import tpu as pltpu
from jax.experimental.pallas import tpu_sc as plsc

_N_NODES = 10000
_N_EDGES = 320000
_N_IN = 128
_N_HID = 128
_E_IN = 16
_EF = 4
_E_HID = 128

_NPC = 2
_EP = _N_EDGES // _NPC
_NW = 32
_C = 40
_E_PER_W = _EP // _NW
_N_CHUNKS = _E_PER_W // _C

_HI = jax.lax.Precision.HIGHEST


def _prep_body(x_ref, w1t_ref, ei_ref, h_ref, src_ref, dst_ref):
    h_ref[...] = jnp.dot(x_ref[...], w1t_ref[...],
                         preferred_element_type=jnp.float32, precision=_HI)
    start = jnp.min(ei_ref[0, :])
    src_ref[...] = jnp.clip(ei_ref[0, :] - start, 0, _N_NODES - 1)
    dst_ref[...] = jnp.clip(ei_ref[1, :] - start, 0, _N_NODES - 1)


def _prep(x, w1t, ei):
    return pl.pallas_call(
        _prep_body,
        out_shape=(
            jax.ShapeDtypeStruct((_N_NODES, _N_HID), jnp.float32),
            jax.ShapeDtypeStruct((_N_EDGES,), jnp.int32),
            jax.ShapeDtypeStruct((_N_EDGES,), jnp.int32),
        ),
    )(x, w1t, ei)


_NBUF = 4
_AHEAD = 2


def _make_gather(piece):
    @functools.partial(
        pl.kernel,
        mesh=plsc.VectorSubcoreMesh(core_axis_name="c", subcore_axis_name="s"),
        out_type=(
            jax.ShapeDtypeStruct((_EP, _N_HID), jnp.float32),
            jax.ShapeDtypeStruct((_EP, _N_HID), jnp.float32),
        ),
        scratch_types=[
            pltpu.VMEM((_E_PER_W,), jnp.int32),
            pltpu.VMEM((_E_PER_W,), jnp.int32),
            pltpu.VMEM((_NBUF, _C, _N_HID), jnp.float32),
            pltpu.VMEM((_NBUF, _C, _N_HID), jnp.float32),
            pltpu.SemaphoreType.DMA((_NBUF,)),
            pltpu.SemaphoreType.DMA((_NBUF,)),
        ],
    )
    def _gather(h_hbm, src_hbm, dst_hbm, xs_hbm, xd_hbm, isl, idl, rs, rd, gsem, osem):
        wid = lax.axis_index("s") * 2 + lax.axis_index("c")
        obase = wid * _E_PER_W
        ebase = piece * _EP + obase

        pltpu.sync_copy(src_hbm.at[pl.ds(ebase, _E_PER_W)], isl)
        pltpu.sync_copy(dst_hbm.at[pl.ds(ebase, _E_PER_W)], idl)

        def start_gather(i, slot):
            pltpu.async_copy(h_hbm.at[isl.at[pl.ds(i * _C, _C)]], rs.at[slot], gsem.at[slot])
            pltpu.async_copy(h_hbm.at[idl.at[pl.ds(i * _C, _C)]], rd.at[slot], gsem.at[slot])

        def wait_gather(slot):
            pltpu.make_async_copy(h_hbm.at[pl.ds(0, _C)], rs.at[slot], gsem.at[slot]).wait()
            pltpu.make_async_copy(h_hbm.at[pl.ds(0, _C)], rd.at[slot], gsem.at[slot]).wait()

        def drain_scatter(slot):
            pltpu.make_async_copy(rs.at[slot], xs_hbm.at[pl.ds(0, _C)], osem.at[slot]).wait()
            pltpu.make_async_copy(rd.at[slot], xd_hbm.at[pl.ds(0, _C)], osem.at[slot]).wait()

        for j in range(_AHEAD):
            start_gather(j, j)

        def body(i, carry):
            p = lax.rem(i, _NBUF)

            @pl.when(i + _AHEAD < _N_CHUNKS)
            def _():
                q = lax.rem(i + _AHEAD, _NBUF)

                @pl.when(i >= _NBUF - _AHEAD)
                def _():
                    drain_scatter(q)

                start_gather(i + _AHEAD, q)

            wait_gather(p)
            off = pl.ds(obase + i * _C, _C)
            pltpu.async_copy(rs.at[p], xs_hbm.at[off], osem.at[p])
            pltpu.async_copy(rd.at[p], xd_hbm.at[off], osem.at[p])
            return carry

        lax.fori_loop(0, _N_CHUNKS, body, 0)
        for slot in range(_NBUF):
            drain_scatter(slot)

    return _gather


_gather_piece = [_make_gather(p) for p in range(_NPC)]


_BLK = 3200
_BPP = _EP // _BLK


def _out_body(xs_ref, xd_ref, ea_ref, ef_ref, wem_ref, wea_ref, wef_ref,
              b2_ref, out_ref):
    xem = jnp.maximum(xs_ref[...] - xd_ref[...], 0.0).astype(jnp.bfloat16)
    acc = jnp.dot(xem, wem_ref[...], preferred_element_type=jnp.float32)
    acc = acc + jnp.dot(ea_ref[...].astype(jnp.bfloat16), wea_ref[...],
                        preferred_element_type=jnp.float32)
    acc = acc + jnp.dot(ef_ref[...].astype(jnp.bfloat16), wef_ref[...],
                        preferred_element_type=jnp.float32)
    out_ref[...] = acc + b2_ref[...]


def _acc_body(prev_ref, xs_ref, xd_ref, ea_ref, ef_ref, wem_ref, wea_ref,
              wef_ref, b2_ref, out_ref):
    del prev_ref
    _out_body(xs_ref, xd_ref, ea_ref, ef_ref, wem_ref, wea_ref, wef_ref,
              b2_ref, out_ref)


def _final_piece(piece, xs, xd, ea, ef, wem, wea, wef, b2row, prev=None):
    off = piece * _BPP
    specs = [
        pl.BlockSpec((_BLK, _N_HID), lambda i: (i, 0)),
        pl.BlockSpec((_BLK, _N_HID), lambda i: (i, 0)),
        pl.BlockSpec((_BLK, _E_IN), lambda i, o=off: (i + o, 0)),
        pl.BlockSpec((_BLK, _EF), lambda i, o=off: (i + o, 0)),
        pl.BlockSpec((_N_HID, _E_HID), lambda i: (0, 0)),
        pl.BlockSpec((_E_IN, _E_HID), lambda i: (0, 0)),
        pl.BlockSpec((_EF, _E_HID), lambda i: (0, 0)),
        pl.BlockSpec((1, _E_HID), lambda i: (0, 0)),
    ]
    out_spec = pl.BlockSpec((_BLK, _E_HID), lambda i, o=off: (i + o, 0))
    out_shape = jax.ShapeDtypeStruct((_N_EDGES, _E_HID), jnp.float32)
    if prev is None:
        return pl.pallas_call(
            _out_body, grid=(_BPP,), in_specs=specs, out_specs=out_spec,
            out_shape=out_shape,
        )(xs, xd, ea, ef, wem, wea, wef, b2row)
    prev_spec = pl.BlockSpec((8, _E_HID), lambda i: (0, 0))
    return pl.pallas_call(
        _acc_body, grid=(_BPP,), in_specs=[prev_spec] + specs,
        out_specs=out_spec, out_shape=out_shape,
        input_output_aliases={0: 0},
    )(prev, xs, xd, ea, ef, wem, wea, wef, b2row)


def kernel(x, edge_index, edge_f, edge_attr, W1, b1, W2, b2):
    del b1
    h, src, dst = _prep(x, W1.T, edge_index)
    wem = W2[:, :_N_HID].T.astype(jnp.bfloat16)
    wea = W2[:, _N_HID:_N_HID + _E_IN].T.astype(jnp.bfloat16)
    wef = W2[:, _N_HID + _E_IN:].T.astype(jnp.bfloat16)
    b2row = b2.reshape(1, _E_HID)
    rows = [_gather_piece[p](h, src, dst) for p in range(_NPC)]
    out = None
    for p, (xs, xd) in enumerate(rows):
        out = _final_piece(p, xs, xd, edge_attr, edge_f, wem, wea, wef, b2row,
                           prev=out)
    return out


# speedup vs baseline: 1.0408x; 1.0017x over previous
import functools

import jax
import jax.numpy as jnp
from jax import lax
from jax.experimental import pallas as pl
from jax.experimental.pallas import tpu as pltpu
from jax.experimental.pallas import tpu_sc as plsc

_N_NODES = 10000
_N_EDGES = 320000
_N_IN = 128
_N_HID = 128
_E_IN = 16
_EF = 4
_E_HID = 128

_NPC = 2
_EP = _N_EDGES // _NPC
_NW = 32
_C = 40
_E_PER_W = _EP // _NW
_N_CHUNKS = _E_PER_W // _C

_HI = jax.lax.Precision.HIGHEST


def _prep_body(x_ref, w1t_ref, ei_ref, h_ref, src_ref, dst_ref):
    h_ref[...] = jnp.dot(x_ref[...], w1t_ref[...],
                         preferred_element_type=jnp.float32, precision=_HI)
    start = jnp.min(ei_ref[0, :])
    src_ref[...] = jnp.clip(ei_ref[0, :] - start, 0, _N_NODES - 1)
    dst_ref[...] = jnp.clip(ei_ref[1, :] - start, 0, _N_NODES - 1)


def _prep(x, w1t, ei):
    return pl.pallas_call(
        _prep_body,
        out_shape=(
            jax.ShapeDtypeStruct((_N_NODES, _N_HID), jnp.float32),
            jax.ShapeDtypeStruct((_N_EDGES,), jnp.int32),
            jax.ShapeDtypeStruct((_N_EDGES,), jnp.int32),
        ),
    )(x, w1t, ei)


_NBUF = 4
_AHEAD = 2


def _make_gather(piece):
    @functools.partial(
        pl.kernel,
        mesh=plsc.VectorSubcoreMesh(core_axis_name="c", subcore_axis_name="s"),
        compiler_params=pltpu.CompilerParams(use_tc_tiling_on_sc=True),
        out_type=(
            jax.ShapeDtypeStruct((_EP, _N_HID), jnp.float32),
            jax.ShapeDtypeStruct((_EP, _N_HID), jnp.float32),
        ),
        scratch_types=[
            pltpu.VMEM((_E_PER_W,), jnp.int32),
            pltpu.VMEM((_E_PER_W,), jnp.int32),
            pltpu.VMEM((_NBUF, _C, _N_HID), jnp.float32),
            pltpu.VMEM((_NBUF, _C, _N_HID), jnp.float32),
            pltpu.SemaphoreType.DMA((_NBUF,)),
            pltpu.SemaphoreType.DMA((_NBUF,)),
        ],
    )
    def _gather(h_hbm, src_hbm, dst_hbm, xs_hbm, xd_hbm, isl, idl, rs, rd, gsem, osem):
        wid = lax.axis_index("s") * 2 + lax.axis_index("c")
        obase = wid * _E_PER_W
        ebase = piece * _EP + obase

        pltpu.sync_copy(src_hbm.at[pl.ds(ebase, _E_PER_W)], isl)
        pltpu.sync_copy(dst_hbm.at[pl.ds(ebase, _E_PER_W)], idl)

        def start_gather(i, slot):
            pltpu.async_copy(h_hbm.at[isl.at[pl.ds(i * _C, _C)]], rs.at[slot], gsem.at[slot])
            pltpu.async_copy(h_hbm.at[idl.at[pl.ds(i * _C, _C)]], rd.at[slot], gsem.at[slot])

        def wait_gather(slot):
            pltpu.make_async_copy(h_hbm.at[pl.ds(0, _C)], rs.at[slot], gsem.at[slot]).wait()
            pltpu.make_async_copy(h_hbm.at[pl.ds(0, _C)], rd.at[slot], gsem.at[slot]).wait()

        def drain_scatter(slot):
            pltpu.make_async_copy(rs.at[slot], xs_hbm.at[pl.ds(0, _C)], osem.at[slot]).wait()
            pltpu.make_async_copy(rd.at[slot], xd_hbm.at[pl.ds(0, _C)], osem.at[slot]).wait()

        for j in range(_AHEAD):
            start_gather(j, j)

        def body(i, carry):
            p = lax.rem(i, _NBUF)

            @pl.when(i + _AHEAD < _N_CHUNKS)
            def _():
                q = lax.rem(i + _AHEAD, _NBUF)

                @pl.when(i >= _NBUF - _AHEAD)
                def _():
                    drain_scatter(q)

                start_gather(i + _AHEAD, q)

            wait_gather(p)
            off = pl.ds(obase + i * _C, _C)
            pltpu.async_copy(rs.at[p], xs_hbm.at[off], osem.at[p])
            pltpu.async_copy(rd.at[p], xd_hbm.at[off], osem.at[p])
            return carry

        lax.fori_loop(0, _N_CHUNKS, body, 0)
        for slot in range(_NBUF):
            drain_scatter(slot)

    return _gather


_gather_piece = [_make_gather(p) for p in range(_NPC)]


_BLK = 3200
_BPP = _EP // _BLK


def _out_body(xs_ref, xd_ref, ea_ref, ef_ref, wem_ref, wea_ref, wef_ref,
              b2_ref, out_ref):
    xem = jnp.maximum(xs_ref[...] - xd_ref[...], 0.0).astype(jnp.bfloat16)
    acc = jnp.dot(xem, wem_ref[...], preferred_element_type=jnp.float32)
    acc = acc + jnp.dot(ea_ref[...].astype(jnp.bfloat16), wea_ref[...],
                        preferred_element_type=jnp.float32)
    acc = acc + jnp.dot(ef_ref[...].astype(jnp.bfloat16), wef_ref[...],
                        preferred_element_type=jnp.float32)
    out_ref[...] = acc + b2_ref[...]


def _acc_body(prev_ref, xs_ref, xd_ref, ea_ref, ef_ref, wem_ref, wea_ref,
              wef_ref, b2_ref, out_ref):
    del prev_ref
    _out_body(xs_ref, xd_ref, ea_ref, ef_ref, wem_ref, wea_ref, wef_ref,
              b2_ref, out_ref)


def _final_piece(piece, xs, xd, ea, ef, wem, wea, wef, b2row, prev=None):
    off = piece * _BPP
    specs = [
        pl.BlockSpec((_BLK, _N_HID), lambda i: (i, 0)),
        pl.BlockSpec((_BLK, _N_HID), lambda i: (i, 0)),
        pl.BlockSpec((_BLK, _E_IN), lambda i, o=off: (i + o, 0)),
        pl.BlockSpec((_BLK, _EF), lambda i, o=off: (i + o, 0)),
        pl.BlockSpec((_N_HID, _E_HID), lambda i: (0, 0)),
        pl.BlockSpec((_E_IN, _E_HID), lambda i: (0, 0)),
        pl.BlockSpec((_EF, _E_HID), lambda i: (0, 0)),
        pl.BlockSpec((1, _E_HID), lambda i: (0, 0)),
    ]
    out_spec = pl.BlockSpec((_BLK, _E_HID), lambda i, o=off: (i + o, 0))
    out_shape = jax.ShapeDtypeStruct((_N_EDGES, _E_HID), jnp.float32)
    if prev is None:
        return pl.pallas_call(
            _out_body, grid=(_BPP,), in_specs=specs, out_specs=out_spec,
            out_shape=out_shape,
        )(xs, xd, ea, ef, wem, wea, wef, b2row)
    prev_spec = pl.BlockSpec((8, _E_HID), lambda i: (0, 0))
    return pl.pallas_call(
        _acc_body, grid=(_BPP,), in_specs=[prev_spec] + specs,
        out_specs=out_spec, out_shape=out_shape,
        input_output_aliases={0: 0},
    )(prev, xs, xd, ea, ef, wem, wea, wef, b2row)


def kernel(x, edge_index, edge_f, edge_attr, W1, b1, W2, b2):
    del b1
    h, src, dst = _prep(x, W1.T, edge_index)
    wem = W2[:, :_N_HID].T.astype(jnp.bfloat16)
    wea = W2[:, _N_HID:_N_HID + _E_IN].T.astype(jnp.bfloat16)
    wef = W2[:, _N_HID + _E_IN:].T.astype(jnp.bfloat16)
    b2row = b2.reshape(1, _E_HID)
    rows = [_gather_piece[p](h, src, dst) for p in range(_NPC)]
    out = None
    for p, (xs, xd) in enumerate(rows):
        out = _final_piece(p, xs, xd, edge_attr, edge_f, wem, wea, wef, b2row,
                           prev=out)
    return out
